# Initial kernel scaffold; baseline (speedup 1.0000x reference)
import jax
import jax.numpy as jnp
from jax.experimental import pallas as pl


def kernel(points, pts_img, bu0, bu1, td0, td1, sa0_s0_w0, sa0_s0_b0, sa0_s0_w1, sa0_s0_b1, sa0_s0_w2, sa0_s0_b2, sa0_s1_w0, sa0_s1_b0, sa0_s1_w1, sa0_s1_b1, sa0_s1_w2, sa0_s1_b2, sa1_s0_w0, sa1_s0_b0, sa1_s0_w1, sa1_s0_b1, sa1_s0_w2, sa1_s0_b2, sa1_s1_w0, sa1_s1_b0, sa1_s1_w1, sa1_s1_b1, sa1_s1_w2, sa1_s1_b2, pwgbu0_w, pwgbu0_b, pwgtd0_w, pwgtd0_b, pwgbu1_w, pwgbu1_b, pwgtd1_w, pwgtd1_b, fp0_w0, fp0_b0, fp0_w1, fp0_b1, fp1_w0, fp1_b0, fp1_w1, fp1_b1):
    raise NotImplementedError("write your pallas kernel here")


# speedup vs baseline: 8.1139x; 8.1139x over previous
import functools
import jax
import jax.numpy as jnp
from jax import lax
from jax.experimental import pallas as pl
from jax.experimental.pallas import tpu as pltpu
from jax.experimental.pallas import tpu_sc as plsc

_B = 2
_N = 8192
_H_IMG, _W_IMG = 384.0, 1280.0
_BM = 128


def _sc_gather(table, idx):
    V, D = table.shape
    Bt = idx.shape[0]
    NW = 32
    b_per_w = Bt // NW
    cb = b_per_w
    while cb * D * 4 > 262144:
        cb //= 2
    nch = b_per_w // cb
    mesh = plsc.VectorSubcoreMesh(core_axis_name="c", subcore_axis_name="s")

    @functools.partial(
        pl.kernel,
        mesh=mesh,
        out_type=jax.ShapeDtypeStruct((Bt, D), jnp.float32),
        scratch_types=[
            pltpu.VMEM((cb,), jnp.int32),
            pltpu.VMEM((cb, D), jnp.float32),
            pltpu.SemaphoreType.DMA,
        ],
    )
    def k(table_hbm, idx_hbm, out_hbm, idx_v, rows_v, sem):
        wid = lax.axis_index("s") * 2 + lax.axis_index("c")
        base = wid * b_per_w
        for i in range(nch):
            off = base + i * cb
            pltpu.sync_copy(idx_hbm.at[pl.ds(off, cb)], idx_v)
            pltpu.async_copy(table_hbm.at[idx_v], rows_v, sem).wait()
            pltpu.sync_copy(rows_v, out_hbm.at[pl.ds(off, cb)])

    return k(table, idx)


def _fps_body(xyzT_ref, piT_ref, oxyz_ref, opi_ref, *, n, npoint):
    X = xyzT_ref[0, 0:1, :]
    Y = xyzT_ref[0, 1:2, :]
    Z = xyzT_ref[0, 2:3, :]
    U = piT_ref[0, 0:1, :]
    Vv = piT_ref[0, 1:2, :]
    iota = lax.broadcasted_iota(jnp.int32, (1, n), 1)

    def body(i, carry):
        dists, sel = carry
        oh = (iota == sel).astype(jnp.float32)
        cx = jnp.sum(oh * X)
        cy = jnp.sum(oh * Y)
        cz = jnp.sum(oh * Z)
        cu = jnp.sum(oh * U)
        cv = jnp.sum(oh * Vv)
        oxyz_ref[0, pl.ds(i, 1), :] = jnp.concatenate(
            [cx.reshape(1, 1), cy.reshape(1, 1), cz.reshape(1, 1)], axis=1)
        opi_ref[0, pl.ds(i, 1), :] = jnp.concatenate(
            [cu.reshape(1, 1), cv.reshape(1, 1)], axis=1)
        d = (X - cx) ** 2 + (Y - cy) ** 2 + (Z - cz) ** 2
        dists = jnp.minimum(dists, d)
        m = jnp.max(dists)
        sel2 = jnp.min(jnp.where(dists == m, iota, n), axis=1, keepdims=True)
        return dists, sel2

    lax.fori_loop(0, npoint, body,
                  (jnp.full((1, n), 1e10, jnp.float32),
                   jnp.zeros((1, 1), jnp.int32)))


def _fps(xyzT, piT, npoint):
    B, _, n = xyzT.shape
    return pl.pallas_call(
        functools.partial(_fps_body, n=n, npoint=npoint),
        grid=(B,),
        in_specs=[
            pl.BlockSpec((1, 3, n), lambda b: (b, 0, 0)),
            pl.BlockSpec((1, 2, n), lambda b: (b, 0, 0)),
        ],
        out_specs=[
            pl.BlockSpec((1, npoint, 3), lambda b: (b, 0, 0)),
            pl.BlockSpec((1, npoint, 2), lambda b: (b, 0, 0)),
        ],
        out_shape=[
            jax.ShapeDtypeStruct((B, npoint, 3), jnp.float32),
            jax.ShapeDtypeStruct((B, npoint, 2), jnp.float32),
        ],
    )(xyzT, piT)


def _select_body(nx_ref, xT_ref, idx_ref, dist_ref, *, n, K):
    b = pl.program_id(0)
    nx = nx_ref[0]
    xT = xT_ref[0]
    aa = jnp.sum(nx * nx, axis=1, keepdims=True)
    bb = jnp.sum(xT * xT, axis=0, keepdims=True)
    ab = jnp.dot(nx, xT, preferred_element_type=jnp.float32)
    D = aa + bb - 2.0 * ab
    iota = lax.broadcasted_iota(jnp.int32, D.shape, 1)
    base = b * n
    for s in range(K):
        m = jnp.min(D, axis=1, keepdims=True)
        cand = jnp.where(D == m, iota, n)
        sel = jnp.min(cand, axis=1, keepdims=True)
        idx_ref[0, :, pl.ds(s, 1)] = sel + base
        dist_ref[0, :, pl.ds(s, 1)] = m
        D = jnp.where(cand == sel, jnp.inf, D)


def _knn_select(nxyz, xT, K):
    B, M, _ = nxyz.shape
    n = xT.shape[2]
    return pl.pallas_call(
        functools.partial(_select_body, n=n, K=K),
        grid=(B, M // _BM),
        in_specs=[
            pl.BlockSpec((1, _BM, 3), lambda b, m: (b, m, 0)),
            pl.BlockSpec((1, 3, n), lambda b, m: (b, 0, 0)),
        ],
        out_specs=[
            pl.BlockSpec((1, _BM, K), lambda b, m: (b, m, 0)),
            pl.BlockSpec((1, _BM, K), lambda b, m: (b, m, 0)),
        ],
        out_shape=[
            jax.ShapeDtypeStruct((B, M, K), jnp.int32),
            jax.ShapeDtypeStruct((B, M, K), jnp.float32),
        ],
    )(nxyz, xT)


def _sa_mlp_body(g_ref, nx_ref, d_ref, w0_ref, b0_ref, w1_ref, b1_ref,
                 w2_ref, b2_ref, out_ref, *, ns, Dp, C, r2):
    g = g_ref[0]
    nx = nx_ref[0]
    ds = d_ref[0]
    w0 = w0_ref[...]
    b0 = b0_ref[...]
    w1 = w1_ref[...]
    b1 = b1_ref[...]
    w2 = w2_ref[...]
    b2 = b2_ref[...]
    raw0 = g[:, 0:Dp]
    acc = None
    for s in range(ns):
        raw = g[:, s * Dp:(s + 1) * Dp]
        valid = ds[:, s:s + 1] <= r2
        row = jnp.where(valid, raw, raw0)
        x = jnp.concatenate([row[:, 0:3] - nx, row[:, 3:3 + C]], axis=1)
        x = jax.nn.relu(jnp.dot(x, w0, preferred_element_type=jnp.float32) + b0)
        x = jax.nn.relu(jnp.dot(x, w1, preferred_element_type=jnp.float32) + b1)
        x = jax.nn.relu(jnp.dot(x, w2, preferred_element_type=jnp.float32) + b2)
        acc = x if acc is None else jnp.maximum(acc, x)
    out_ref[0] = acc


def _sa_mlp(G, nxyz, dist, ws, ns, Dp, C, r2):
    B, M, _ = nxyz.shape
    K = dist.shape[2]
    (w0, b0, w1, b1, w2, b2) = ws
    Cout = w2.shape[1]
    full = lambda a: pl.BlockSpec(a.shape, lambda b, m: (0,) * a.ndim)
    return pl.pallas_call(
        functools.partial(_sa_mlp_body, ns=ns, Dp=Dp, C=C, r2=r2),
        grid=(B, M // _BM),
        in_specs=[
            pl.BlockSpec((1, _BM, ns * Dp), lambda b, m: (b, m, 0)),
            pl.BlockSpec((1, _BM, 3), lambda b, m: (b, m, 0)),
            pl.BlockSpec((1, _BM, K), lambda b, m: (b, m, 0)),
            full(w0), full(b0), full(w1), full(b1), full(w2), full(b2),
        ],
        out_specs=pl.BlockSpec((1, _BM, Cout), lambda b, m: (b, m, 0)),
        out_shape=jax.ShapeDtypeStruct((B, M, Cout), jnp.float32),
    )(G, nxyz, dist, w0, b0, w1, b1, w2, b2)


def _fp_mlp_body(g_ref, d3_ref, skip_ref, w0_ref, b0_ref, w1_ref, b1_ref,
                 out_ref, *, C):
    g = g_ref[0]
    d3 = d3_ref[0]
    w = 1.0 / (jnp.maximum(d3, 0.0) + 1e-8)
    wn = w / jnp.sum(w, axis=1, keepdims=True)
    interp = (g[:, 0:C] * wn[:, 0:1] + g[:, C:2 * C] * wn[:, 1:2]
              + g[:, 2 * C:3 * C] * wn[:, 2:3])
    x = jnp.concatenate([interp, skip_ref[0]], axis=1)
    x = jax.nn.relu(jnp.dot(x, w0_ref[...],
                            preferred_element_type=jnp.float32) + b0_ref[...])
    x = jax.nn.relu(jnp.dot(x, w1_ref[...],
                            preferred_element_type=jnp.float32) + b1_ref[...])
    out_ref[0] = x


def _fp_mlp(G, d3, skip, ws, C):
    B, M, _ = d3.shape
    (w0, b0, w1, b1) = ws
    Cs = skip.shape[2]
    Cout = w1.shape[1]
    full = lambda a: pl.BlockSpec(a.shape, lambda b, m: (0,) * a.ndim)
    return pl.pallas_call(
        functools.partial(_fp_mlp_body, C=C),
        grid=(B, M // _BM),
        in_specs=[
            pl.BlockSpec((1, _BM, 3 * C), lambda b, m: (b, m, 0)),
            pl.BlockSpec((1, _BM, 3), lambda b, m: (b, m, 0)),
            pl.BlockSpec((1, _BM, Cs), lambda b, m: (b, m, 0)),
            full(w0), full(b0), full(w1), full(b1),
        ],
        out_specs=pl.BlockSpec((1, _BM, Cout), lambda b, m: (b, m, 0)),
        out_shape=jax.ShapeDtypeStruct((B, M, Cout), jnp.float32),
    )(G, d3, skip, w0, b0, w1, b1)


def _pwg_prep_body(pts_ref, idx_ref, w_ref, *, H, W):
    b = pl.program_id(0)
    g = pts_ref[0]
    x = (g[:, 0:1] + 1.0) * W / 2.0 - 0.5
    y = (g[:, 1:2] + 1.0) * H / 2.0 - 0.5
    x0 = jnp.floor(x)
    y0 = jnp.floor(y)
    wx = x - x0
    wy = y - y0
    tw = [(1 - wx) * (1 - wy), wx * (1 - wy), (1 - wx) * wy, wx * wy]
    for t, (dx, dy) in enumerate([(0, 0), (1, 0), (0, 1), (1, 1)]):
        xi = jnp.clip((x0 + dx).astype(jnp.int32), 0, W - 1)
        yi = jnp.clip((y0 + dy).astype(jnp.int32), 0, H - 1)
        idx_ref[0, :, pl.ds(t, 1)] = (b * H + yi) * W + xi
        w_ref[0, :, pl.ds(t, 1)] = tw[t]


def _pwg_prep(pts, H, W):
    B, M, _ = pts.shape
    return pl.pallas_call(
        functools.partial(_pwg_prep_body, H=H, W=W),
        grid=(B, M // _BM),
        in_specs=[pl.BlockSpec((1, _BM, 2), lambda b, m: (b, m, 0))],
        out_specs=[
            pl.BlockSpec((1, _BM, 4), lambda b, m: (b, m, 0)),
            pl.BlockSpec((1, _BM, 4), lambda b, m: (b, m, 0)),
        ],
        out_shape=[
            jax.ShapeDtypeStruct((B, M, 4), jnp.int32),
            jax.ShapeDtypeStruct((B, M, 4), jnp.float32),
        ],
    )(pts)


def _pwg_gate_body(g_ref, tw_ref, pf_ref, w_ref, b_ref, out_ref, *, C, Cp):
    g = g_ref[0]
    tw = tw_ref[0]
    samp = (g[:, 0:C] * tw[:, 0:1] + g[:, Cp:Cp + C] * tw[:, 1:2]
            + g[:, 2 * Cp:2 * Cp + C] * tw[:, 2:3]
            + g[:, 3 * Cp:3 * Cp + C] * tw[:, 3:4])
    gate = jax.nn.sigmoid(
        jnp.dot(samp, w_ref[...], preferred_element_type=jnp.float32)
        + b_ref[...])
    out_ref[0] = pf_ref[0] * gate


def _pwg_gate(G, tw, pfeats, w, b2d, Cp):
    B, M, Cf = pfeats.shape
    C = w.shape[0]
    full = lambda a: pl.BlockSpec(a.shape, lambda b, m: (0,) * a.ndim)
    return pl.pallas_call(
        functools.partial(_pwg_gate_body, C=C, Cp=Cp),
        grid=(B, M // _BM),
        in_specs=[
            pl.BlockSpec((1, _BM, 4 * Cp), lambda b, m: (b, m, 0)),
            pl.BlockSpec((1, _BM, 4), lambda b, m: (b, m, 0)),
            pl.BlockSpec((1, _BM, Cf), lambda b, m: (b, m, 0)),
            full(w), full(b2d),
        ],
        out_specs=pl.BlockSpec((1, _BM, Cf), lambda b, m: (b, m, 0)),
        out_shape=jax.ShapeDtypeStruct((B, M, Cf), jnp.float32),
    )(G, tw, pfeats, w, b2d)


def _pwg_fuse(pts, img, pfeats, w, b):
    B, C, H, W = img.shape
    Cp = max(C, 128)
    ti, tw = _pwg_prep(pts, H, W)
    flat = jnp.transpose(img, (0, 2, 3, 1)).reshape(B * H * W, C)
    G = _sc_gather(_pad_cols(flat, Cp), ti.reshape(-1))
    M = pts.shape[1]
    return _pwg_gate(G.reshape(B, M, 4 * Cp), tw, pfeats, w,
                     b.reshape(1, -1), Cp)


def _pad_cols(a, Dp):
    return jnp.pad(a, ((0, 0), (0, Dp - a.shape[1])))


def kernel(points, pts_img, bu0, bu1, td0, td1,
           sa0_s0_w0, sa0_s0_b0, sa0_s0_w1, sa0_s0_b1, sa0_s0_w2, sa0_s0_b2,
           sa0_s1_w0, sa0_s1_b0, sa0_s1_w1, sa0_s1_b1, sa0_s1_w2, sa0_s1_b2,
           sa1_s0_w0, sa1_s0_b0, sa1_s0_w1, sa1_s0_b1, sa1_s0_w2, sa1_s0_b2,
           sa1_s1_w0, sa1_s1_b0, sa1_s1_w1, sa1_s1_b1, sa1_s1_w2, sa1_s1_b2,
           pwgbu0_w, pwgbu0_b, pwgtd0_w, pwgtd0_b,
           pwgbu1_w, pwgbu1_b, pwgtd1_w, pwgtd1_b,
           fp0_w0, fp0_b0, fp0_w1, fp0_b1,
           fp1_w0, fp1_b0, fp1_w1, fp1_b1):
    r2 = lambda b: b.reshape(1, -1)
    xyz = points[:, 1:4].reshape(_B, _N, 3)
    feats = points[:, 4:].reshape(_B, _N, 1)
    pi = pts_img[:, 1:].reshape(_B, _N, 2)
    pi = jnp.stack([2.0 * (pi[..., 0] / _W_IMG) - 1.0,
                    2.0 * (pi[..., 1] / _H_IMG) - 1.0], axis=-1)

    xyzT0 = jnp.transpose(xyz, (0, 2, 1))
    piT0 = jnp.transpose(pi, (0, 2, 1))
    nxyz0, npi0 = _fps(xyzT0, piT0, 2048)
    idx0, dist0 = _knn_select(nxyz0, xyzT0, 32)
    table0 = _pad_cols(jnp.concatenate([xyz, feats], -1).reshape(_B * _N, 4),
                       128)
    G0 = _sc_gather(table0, idx0.reshape(-1)).reshape(_B, 2048, 32 * 128)
    f00 = _sa_mlp(G0[:, :, :16 * 128], nxyz0, dist0,
                  (sa0_s0_w0, r2(sa0_s0_b0), sa0_s0_w1, r2(sa0_s0_b1),
                   sa0_s0_w2, r2(sa0_s0_b2)), 16, 128, 1, 0.25)
    f01 = _sa_mlp(G0, nxyz0, dist0,
                  (sa0_s1_w0, r2(sa0_s1_b0), sa0_s1_w1, r2(sa0_s1_b1),
                   sa0_s1_w2, r2(sa0_s1_b2)), 32, 128, 1, 1.0)
    nf0 = jnp.concatenate([f00, f01], -1)
    nf0 = _pwg_fuse(npi0, bu0, nf0, pwgbu0_w, pwgbu0_b)

    xyzT1 = jnp.transpose(nxyz0, (0, 2, 1))
    npiT1 = jnp.transpose(npi0, (0, 2, 1))
    nxyz1, npi1 = _fps(xyzT1, npiT1, 512)
    idx1, dist1 = _knn_select(nxyz1, xyzT1, 32)
    table1 = _pad_cols(jnp.concatenate([nxyz0, nf0], -1).reshape(_B * 2048, 99),
                       128)
    G1 = _sc_gather(table1, idx1.reshape(-1)).reshape(_B, 512, 32 * 128)
    f10 = _sa_mlp(G1[:, :, :16 * 128], nxyz1, dist1,
                  (sa1_s0_w0, r2(sa1_s0_b0), sa1_s0_w1, r2(sa1_s0_b1),
                   sa1_s0_w2, r2(sa1_s0_b2)), 16, 128, 96, 1.0)
    f11 = _sa_mlp(G1, nxyz1, dist1,
                  (sa1_s1_w0, r2(sa1_s1_b0), sa1_s1_w1, r2(sa1_s1_b1),
                   sa1_s1_w2, r2(sa1_s1_b2)), 32, 128, 96, 4.0)
    nf1 = jnp.concatenate([f10, f11], -1)
    nf1 = _pwg_fuse(npi1, bu1, nf1, pwgbu1_w, pwgbu1_b)

    fidx1, fd1 = _knn_select(nxyz0, jnp.transpose(nxyz1, (0, 2, 1)), 3)
    Gf1 = _sc_gather(nf1.reshape(_B * 512, 256),
                     fidx1.reshape(-1)).reshape(_B, 2048, 3 * 256)
    x1 = _fp_mlp(Gf1, fd1, nf0,
                 (fp1_w0, r2(fp1_b0), fp1_w1, r2(fp1_b1)), 256)
    x1 = _pwg_fuse(npi0, td1, x1, pwgtd1_w, pwgtd1_b)

    fidx0, fd0 = _knn_select(xyz, xyzT1, 3)
    Gf0 = _sc_gather(x1.reshape(_B * 2048, 256),
                     fidx0.reshape(-1)).reshape(_B, _N, 3 * 256)
    x0 = _fp_mlp(Gf0, fd0, feats,
                 (fp0_w0, r2(fp0_b0), fp0_w1, r2(fp0_b1)), 256)
    x0 = _pwg_fuse(pi, td0, x0, pwgtd0_w, pwgtd0_b)

    point_features = x0.reshape(-1, 128)
    point_coords = jnp.concatenate([points[:, 0:1], xyz.reshape(-1, 3)], axis=1)
    return point_features, point_coords
